# baseline (speedup 1.0000x reference)
import functools

import jax
import jax.numpy as jnp
from jax.experimental import pallas as pl
from jax.experimental.pallas import tpu as pltpu
from jax.experimental.pallas import tpu_sc as plsc

_B = 320000
_C = 128
_H = 128
_NNZ = 4096
_BLK = 16000

_NS = 16
_TOTAL = 2 * _NNZ
_PER_TILE = _TOTAL // _NS
_CHUNK = 128
_NCHUNK = _PER_TILE // _CHUNK
_WWORDS = 2 * _C * _H


def _densify_sc(flat_hbm, vals_hbm, zeros_hbm, out_hbm,
                idx_v, vals_v, shared):
    sid = jax.lax.axis_index("s")
    cid = jax.lax.axis_index("c")
    base = sid * _PER_TILE
    for j in range(_NCHUNK):
        pltpu.sync_copy(flat_hbm.at[pl.ds(base + j * _CHUNK, _CHUNK)],
                        idx_v.at[j])
        pltpu.sync_copy(vals_hbm.at[pl.ds(base + j * _CHUNK, _CHUNK)],
                        vals_v.at[j])

    @pl.when(sid == 0)
    def _():
        pltpu.sync_copy(zeros_hbm, shared)

    plsc.subcore_barrier()

    for j in range(_NCHUNK):
        pltpu.sync_copy(vals_v.at[j], shared.at[idx_v.at[j]], add=True)

    plsc.subcore_barrier()

    @pl.when(jnp.logical_and(sid == 0, cid == 0))
    def _():
        pltpu.sync_copy(shared, out_hbm)


def _densify(flat_all, vals_all, zeros):
    mesh = plsc.VectorSubcoreMesh(core_axis_name="c", subcore_axis_name="s",
                                  num_cores=2, num_subcores=_NS)
    run = functools.partial(
        pl.kernel,
        out_type=jax.ShapeDtypeStruct((_WWORDS,), jnp.float32),
        mesh=mesh,
        scratch_types=[
            pltpu.VMEM((_NCHUNK, _CHUNK), jnp.int32),
            pltpu.VMEM((_NCHUNK, _CHUNK), jnp.float32),
            pltpu.VMEM_SHARED((_WWORDS,), jnp.float32),
        ],
    )(_densify_sc)
    return run(flat_all, vals_all, zeros)


def _fused_body(wt_in_ref, b_in_ref, wt_out_ref, b_out_ref, x_ref, o_ref):
    x = x_ref[...]
    h = jnp.dot(x, wt_in_ref[...], preferred_element_type=jnp.float32)
    h = jnp.maximum(h + b_in_ref[0:1, :], 0.0)
    o = jnp.dot(h, wt_out_ref[...], preferred_element_type=jnp.float32)
    o_ref[...] = o + b_out_ref[0:1, :] + x


def kernel(x, w_in_vals, b_in, w_out_vals, b_out, in_idx, out_idx):
    flat_in = in_idx[1] * _H + in_idx[0]
    flat_out = _C * _H + out_idx[1] * _C + out_idx[0]
    flat_all = jnp.concatenate([flat_in, flat_out])
    vals_all = jnp.concatenate([w_in_vals, w_out_vals])
    zeros = jnp.zeros((_WWORDS,), jnp.float32)

    w = _densify(flat_all, vals_all, zeros)
    wt_in = w[: _C * _H].reshape(_C, _H)
    wt_out = w[_C * _H:].reshape(_H, _C)

    grid = (_B // _BLK,)
    out = pl.pallas_call(
        _fused_body,
        grid=grid,
        in_specs=[
            pl.BlockSpec((_C, _H), lambda i: (0, 0)),
            pl.BlockSpec((1, _H), lambda i: (0, 0)),
            pl.BlockSpec((_H, _C), lambda i: (0, 0)),
            pl.BlockSpec((1, _C), lambda i: (0, 0)),
            pl.BlockSpec((_BLK, _C), lambda i: (i, 0)),
        ],
        out_specs=pl.BlockSpec((_BLK, _C), lambda i: (i, 0)),
        out_shape=jax.ShapeDtypeStruct((_B, _C), jnp.float32),
    )(
        wt_in,
        b_in.reshape(1, _H),
        wt_out,
        b_out.reshape(1, _C),
        x,
    )
    return out


# speedup vs baseline: 1.0689x; 1.0689x over previous
import functools

import jax
import jax.numpy as jnp
from jax.experimental import pallas as pl
from jax.experimental.pallas import tpu as pltpu
from jax.experimental.pallas import tpu_sc as plsc

_B = 320000
_C = 128
_H = 128
_NNZ = 4096
_BLK = 16000

_NS = 16
_LANES = 16
_PER_TILE = _NNZ // _NS
_CHUNK = 128
_NCHUNK = _PER_TILE // _CHUNK
_WW = _C * _H
_ZCHUNK = _WW // _NS


def _densify_sc(in_idx, in_vals, out_idx, out_vals, zeros,
                wt_in_out, wt_out_out,
                rows_v, cols_v, vals_v, flat_v, sh_in, sh_out, sem):
    sid = jax.lax.axis_index("s")
    cid = jax.lax.axis_index("c")

    copies = []
    for j in range(_NCHUNK):
        off = sid * _PER_TILE + j * _CHUNK
        copies.append(pltpu.async_copy(
            in_idx.at[0, pl.ds(off, _CHUNK)], rows_v.at[j], sem))
        copies.append(pltpu.async_copy(
            in_idx.at[1, pl.ds(off, _CHUNK)], cols_v.at[j], sem))
        copies.append(pltpu.async_copy(
            in_vals.at[pl.ds(off, _CHUNK)], vals_v.at[j], sem))
        copies.append(pltpu.async_copy(
            out_idx.at[0, pl.ds(off, _CHUNK)], rows_v.at[_NCHUNK + j], sem))
        copies.append(pltpu.async_copy(
            out_idx.at[1, pl.ds(off, _CHUNK)], cols_v.at[_NCHUNK + j], sem))
        copies.append(pltpu.async_copy(
            out_vals.at[pl.ds(off, _CHUNK)], vals_v.at[_NCHUNK + j], sem))
    zoff = sid * _ZCHUNK
    copies.append(pltpu.async_copy(
        zeros.at[pl.ds(zoff, _ZCHUNK)], sh_in.at[pl.ds(zoff, _ZCHUNK)], sem))
    copies.append(pltpu.async_copy(
        zeros.at[pl.ds(zoff, _ZCHUNK)], sh_out.at[pl.ds(zoff, _ZCHUNK)], sem))
    for c in copies:
        c.wait()

    for j in range(2 * _NCHUNK):
        for k in range(_CHUNK // _LANES):
            s = pl.ds(k * _LANES, _LANES)
            flat_v[j, s] = cols_v[j, s] * _H + rows_v[j, s]

    plsc.subcore_barrier()

    adds = []
    for j in range(_NCHUNK):
        adds.append(pltpu.async_copy(
            vals_v.at[j], sh_in.at[flat_v.at[j]], sem, add=True))
        adds.append(pltpu.async_copy(
            vals_v.at[_NCHUNK + j], sh_out.at[flat_v.at[_NCHUNK + j]], sem,
            add=True))
    for c in adds:
        c.wait()

    plsc.subcore_barrier()

    @pl.when(jnp.logical_and(sid == 0, cid == 0))
    def _():
        pltpu.sync_copy(sh_in, wt_in_out)

    @pl.when(jnp.logical_and(sid == 0, cid == 1))
    def _():
        pltpu.sync_copy(sh_out, wt_out_out)


def _densify(in_idx, in_vals, out_idx, out_vals, zeros):
    mesh = plsc.VectorSubcoreMesh(core_axis_name="c", subcore_axis_name="s")
    run = functools.partial(
        pl.kernel,
        out_type=[
            jax.ShapeDtypeStruct((_WW,), jnp.float32),
            jax.ShapeDtypeStruct((_WW,), jnp.float32),
        ],
        mesh=mesh,
        scratch_types=[
            pltpu.VMEM((2 * _NCHUNK, _CHUNK), jnp.int32),
            pltpu.VMEM((2 * _NCHUNK, _CHUNK), jnp.int32),
            pltpu.VMEM((2 * _NCHUNK, _CHUNK), jnp.float32),
            pltpu.VMEM((2 * _NCHUNK, _CHUNK), jnp.int32),
            pltpu.VMEM_SHARED((_WW,), jnp.float32),
            pltpu.VMEM_SHARED((_WW,), jnp.float32),
            pltpu.SemaphoreType.DMA,
        ],
    )(_densify_sc)
    return run(in_idx, in_vals, out_idx, out_vals, zeros)


def _fused_body(wt_in_ref, b_in_ref, wt_out_ref, b_out_ref, x_ref, o_ref):
    x = x_ref[...]
    h = jnp.dot(x, wt_in_ref[...], preferred_element_type=jnp.float32)
    h = jnp.maximum(h + b_in_ref[0:1, :], 0.0)
    o = jnp.dot(h, wt_out_ref[...], preferred_element_type=jnp.float32)
    o_ref[...] = o + b_out_ref[0:1, :] + x


def kernel(x, w_in_vals, b_in, w_out_vals, b_out, in_idx, out_idx):
    zeros = jnp.zeros((_WW,), jnp.float32)
    wt_in_flat, wt_out_flat = _densify(
        in_idx, w_in_vals, out_idx, w_out_vals, zeros)
    wt_in = wt_in_flat.reshape(_C, _H)
    wt_out = wt_out_flat.reshape(_H, _C)

    grid = (_B // _BLK,)
    out = pl.pallas_call(
        _fused_body,
        grid=grid,
        in_specs=[
            pl.BlockSpec((_C, _H), lambda i: (0, 0)),
            pl.BlockSpec((1, _H), lambda i: (0, 0)),
            pl.BlockSpec((_H, _C), lambda i: (0, 0)),
            pl.BlockSpec((1, _C), lambda i: (0, 0)),
            pl.BlockSpec((_BLK, _C), lambda i: (i, 0)),
        ],
        out_specs=pl.BlockSpec((_BLK, _C), lambda i: (i, 0)),
        out_shape=jax.ShapeDtypeStruct((_B, _C), jnp.float32),
    )(
        wt_in,
        b_in.reshape(1, _H),
        wt_out,
        b_out.reshape(1, _C),
        x,
    )
    return out


# speedup vs baseline: 1.0816x; 1.0119x over previous
import functools

import jax
import jax.numpy as jnp
from jax.experimental import pallas as pl
from jax.experimental.pallas import tpu as pltpu
from jax.experimental.pallas import tpu_sc as plsc

_B = 320000
_C = 128
_H = 128
_NNZ = 4096
_BLK = 16000
_NBOOT = 3
_N1 = _NBOOT * _BLK

_NS = 16
_LANES = 16
_PER_TILE = _NNZ // _NS
_CHUNK = 128
_NCHUNK = _PER_TILE // _CHUNK
_WW = _C * _H
_ZCHUNK = _WW // _NS


def _densify_sc(in_idx, in_vals, out_idx, out_vals, zeros,
                wt_in_out, wt_out_out,
                rows_v, cols_v, vals_v, flat_v, sh_in, sh_out, sem):
    sid = jax.lax.axis_index("s")
    cid = jax.lax.axis_index("c")

    copies = []
    for j in range(_NCHUNK):
        off = sid * _PER_TILE + j * _CHUNK
        copies.append(pltpu.async_copy(
            in_idx.at[0, pl.ds(off, _CHUNK)], rows_v.at[j], sem))
        copies.append(pltpu.async_copy(
            in_idx.at[1, pl.ds(off, _CHUNK)], cols_v.at[j], sem))
        copies.append(pltpu.async_copy(
            in_vals.at[pl.ds(off, _CHUNK)], vals_v.at[j], sem))
        copies.append(pltpu.async_copy(
            out_idx.at[0, pl.ds(off, _CHUNK)], rows_v.at[_NCHUNK + j], sem))
        copies.append(pltpu.async_copy(
            out_idx.at[1, pl.ds(off, _CHUNK)], cols_v.at[_NCHUNK + j], sem))
        copies.append(pltpu.async_copy(
            out_vals.at[pl.ds(off, _CHUNK)], vals_v.at[_NCHUNK + j], sem))
    zoff = sid * _ZCHUNK
    copies.append(pltpu.async_copy(
        zeros.at[pl.ds(zoff, _ZCHUNK)], sh_in.at[pl.ds(zoff, _ZCHUNK)], sem))
    copies.append(pltpu.async_copy(
        zeros.at[pl.ds(zoff, _ZCHUNK)], sh_out.at[pl.ds(zoff, _ZCHUNK)], sem))
    for c in copies:
        c.wait()

    for j in range(2 * _NCHUNK):
        for k in range(_CHUNK // _LANES):
            s = pl.ds(k * _LANES, _LANES)
            flat_v[j, s] = cols_v[j, s] * _H + rows_v[j, s]

    plsc.subcore_barrier()

    adds = []
    for j in range(_NCHUNK):
        adds.append(pltpu.async_copy(
            vals_v.at[j], sh_in.at[flat_v.at[j]], sem, add=True))
        adds.append(pltpu.async_copy(
            vals_v.at[_NCHUNK + j], sh_out.at[flat_v.at[_NCHUNK + j]], sem,
            add=True))
    for c in adds:
        c.wait()

    plsc.subcore_barrier()

    @pl.when(jnp.logical_and(sid == 0, cid == 0))
    def _():
        pltpu.sync_copy(sh_in, wt_in_out)

    @pl.when(jnp.logical_and(sid == 0, cid == 1))
    def _():
        pltpu.sync_copy(sh_out, wt_out_out)


def _densify(in_idx, in_vals, out_idx, out_vals, zeros):
    mesh = plsc.VectorSubcoreMesh(core_axis_name="c", subcore_axis_name="s")
    run = functools.partial(
        pl.kernel,
        out_type=[
            jax.ShapeDtypeStruct((_WW,), jnp.float32),
            jax.ShapeDtypeStruct((_WW,), jnp.float32),
        ],
        mesh=mesh,
        scratch_types=[
            pltpu.VMEM((2 * _NCHUNK, _CHUNK), jnp.int32),
            pltpu.VMEM((2 * _NCHUNK, _CHUNK), jnp.int32),
            pltpu.VMEM((2 * _NCHUNK, _CHUNK), jnp.float32),
            pltpu.VMEM((2 * _NCHUNK, _CHUNK), jnp.int32),
            pltpu.VMEM_SHARED((_WW,), jnp.float32),
            pltpu.VMEM_SHARED((_WW,), jnp.float32),
            pltpu.SemaphoreType.DMA,
        ],
    )(_densify_sc)
    return run(in_idx, in_vals, out_idx, out_vals, zeros)


def _boot_body(in_idx_ref, in_vals_ref, b_in_ref, out_idx_ref, out_vals_ref,
               b_out_ref, x_ref, o_ref, wt_in_ref, wt_out_ref):
    @pl.when(pl.program_id(0) == 0)
    def _():
        rows_i = in_idx_ref[0:1, :]
        cols_i = in_idx_ref[1:2, :]
        vals_i = in_vals_ref[0:1, :]
        cmat = jnp.where(
            jax.lax.broadcasted_iota(jnp.int32, (_C, _NNZ), 0) == cols_i,
            vals_i, 0.0)
        rmat = jnp.where(
            jax.lax.broadcasted_iota(jnp.int32, (_H, _NNZ), 0) == rows_i,
            1.0, 0.0)
        wt_in_ref[...] = jax.lax.dot_general(
            cmat, rmat, (((1,), (1,)), ((), ())),
            preferred_element_type=jnp.float32)

        rows_o = out_idx_ref[0:1, :]
        cols_o = out_idx_ref[1:2, :]
        vals_o = out_vals_ref[0:1, :]
        hmat = jnp.where(
            jax.lax.broadcasted_iota(jnp.int32, (_H, _NNZ), 0) == cols_o,
            vals_o, 0.0)
        cmat2 = jnp.where(
            jax.lax.broadcasted_iota(jnp.int32, (_C, _NNZ), 0) == rows_o,
            1.0, 0.0)
        wt_out_ref[...] = jax.lax.dot_general(
            hmat, cmat2, (((1,), (1,)), ((), ())),
            preferred_element_type=jnp.float32)

    x = x_ref[...]
    h = jnp.dot(x, wt_in_ref[...], preferred_element_type=jnp.float32)
    h = jnp.maximum(h + b_in_ref[0:1, :], 0.0)
    o = jnp.dot(h, wt_out_ref[...], preferred_element_type=jnp.float32)
    o_ref[...] = o + b_out_ref[0:1, :] + x


def _main_body(prev_ref, wt_in_ref, b_in_ref, wt_out_ref, b_out_ref,
               x_ref, o_ref):
    del prev_ref
    x = x_ref[...]
    h = jnp.dot(x, wt_in_ref[...], preferred_element_type=jnp.float32)
    h = jnp.maximum(h + b_in_ref[0:1, :], 0.0)
    o = jnp.dot(h, wt_out_ref[...], preferred_element_type=jnp.float32)
    o_ref[...] = o + b_out_ref[0:1, :] + x


def kernel(x, w_in_vals, b_in, w_out_vals, b_out, in_idx, out_idx):
    zeros = jnp.zeros((_WW,), jnp.float32)
    b_in2 = b_in.reshape(1, _H)
    b_out2 = b_out.reshape(1, _C)

    wt_in_flat, wt_out_flat = _densify(
        in_idx, w_in_vals, out_idx, w_out_vals, zeros)
    wt_in = wt_in_flat.reshape(_C, _H)
    wt_out = wt_out_flat.reshape(_H, _C)

    boot = pl.pallas_call(
        _boot_body,
        grid=(_NBOOT,),
        in_specs=[
            pl.BlockSpec((2, _NNZ), lambda i: (0, 0)),
            pl.BlockSpec((1, _NNZ), lambda i: (0, 0)),
            pl.BlockSpec((1, _H), lambda i: (0, 0)),
            pl.BlockSpec((2, _NNZ), lambda i: (0, 0)),
            pl.BlockSpec((1, _NNZ), lambda i: (0, 0)),
            pl.BlockSpec((1, _C), lambda i: (0, 0)),
            pl.BlockSpec((_BLK, _C), lambda i: (i, 0)),
        ],
        out_specs=pl.BlockSpec((_BLK, _C), lambda i: (i, 0)),
        out_shape=jax.ShapeDtypeStruct((_B, _C), jnp.float32),
        scratch_shapes=[
            pltpu.VMEM((_C, _H), jnp.float32),
            pltpu.VMEM((_H, _C), jnp.float32),
        ],
    )(
        in_idx,
        w_in_vals.reshape(1, _NNZ),
        b_in2,
        out_idx,
        w_out_vals.reshape(1, _NNZ),
        b_out2,
        x,
    )

    out = pl.pallas_call(
        _main_body,
        grid=((_B - _N1) // _BLK,),
        in_specs=[
            pl.BlockSpec(memory_space=pl.ANY),
            pl.BlockSpec((_C, _H), lambda i: (0, 0)),
            pl.BlockSpec((1, _H), lambda i: (0, 0)),
            pl.BlockSpec((_H, _C), lambda i: (0, 0)),
            pl.BlockSpec((1, _C), lambda i: (0, 0)),
            pl.BlockSpec((_BLK, _C), lambda i: (i + _NBOOT, 0)),
        ],
        out_specs=pl.BlockSpec((_BLK, _C), lambda i: (i + _NBOOT, 0)),
        out_shape=jax.ShapeDtypeStruct((_B, _C), jnp.float32),
        input_output_aliases={0: 0},
    )(
        boot,
        wt_in,
        b_in2,
        wt_out,
        b_out2,
        x,
    )
    return out


# speedup vs baseline: 1.1080x; 1.0244x over previous
import functools

import jax
import jax.numpy as jnp
from jax.experimental import pallas as pl
from jax.experimental.pallas import tpu as pltpu
from jax.experimental.pallas import tpu_sc as plsc

_B = 320000
_C = 128
_H = 128
_NNZ = 4096
_BLK = 20000
_NBOOT = 2
_N1 = _NBOOT * _BLK

_NS = 16
_LANES = 16
_PER_TILE = _NNZ // _NS
_CHUNK = 128
_NCHUNK = _PER_TILE // _CHUNK
_WW = _C * _H
_ZCHUNK = _WW // _NS


def _densify_sc(in_idx, in_vals, out_idx, out_vals, zeros,
                wt_in_out, wt_out_out,
                rows_v, cols_v, vals_v, flat_v, sh_in, sh_out, sem):
    sid = jax.lax.axis_index("s")
    cid = jax.lax.axis_index("c")

    off = sid * _PER_TILE
    copies = [
        pltpu.async_copy(in_idx.at[0, pl.ds(off, _PER_TILE)],
                         rows_v.at[pl.ds(0, _PER_TILE)], sem),
        pltpu.async_copy(in_idx.at[1, pl.ds(off, _PER_TILE)],
                         cols_v.at[pl.ds(0, _PER_TILE)], sem),
        pltpu.async_copy(in_vals.at[pl.ds(off, _PER_TILE)],
                         vals_v.at[pl.ds(0, _PER_TILE)], sem),
        pltpu.async_copy(out_idx.at[0, pl.ds(off, _PER_TILE)],
                         rows_v.at[pl.ds(_PER_TILE, _PER_TILE)], sem),
        pltpu.async_copy(out_idx.at[1, pl.ds(off, _PER_TILE)],
                         cols_v.at[pl.ds(_PER_TILE, _PER_TILE)], sem),
        pltpu.async_copy(out_vals.at[pl.ds(off, _PER_TILE)],
                         vals_v.at[pl.ds(_PER_TILE, _PER_TILE)], sem),
    ]
    zoff = sid * _ZCHUNK
    copies.append(pltpu.async_copy(
        zeros.at[pl.ds(zoff, _ZCHUNK)], sh_in.at[pl.ds(zoff, _ZCHUNK)], sem))
    copies.append(pltpu.async_copy(
        zeros.at[pl.ds(zoff, _ZCHUNK)], sh_out.at[pl.ds(zoff, _ZCHUNK)], sem))
    for c in copies:
        c.wait()

    def _flat_step(i, _):
        s = pl.ds(i * _LANES, _LANES)
        j = i // (_CHUNK // _LANES)
        t = pl.ds((i % (_CHUNK // _LANES)) * _LANES, _LANES)
        flat_v[j, t] = cols_v[s] * _H + rows_v[s]
        return 0

    jax.lax.fori_loop(0, 2 * _PER_TILE // _LANES, _flat_step, 0)

    plsc.subcore_barrier()

    adds = []
    for j in range(_NCHUNK):
        adds.append(pltpu.async_copy(
            vals_v.at[pl.ds(j * _CHUNK, _CHUNK)],
            sh_in.at[flat_v.at[j]], sem, add=True))
        adds.append(pltpu.async_copy(
            vals_v.at[pl.ds((_NCHUNK + j) * _CHUNK, _CHUNK)],
            sh_out.at[flat_v.at[_NCHUNK + j]], sem, add=True))
    for c in adds:
        c.wait()

    plsc.subcore_barrier()

    del cid
    @pl.when(sid == 0)
    def _():
        pltpu.sync_copy(sh_in, wt_in_out)

    @pl.when(sid == 1)
    def _():
        pltpu.sync_copy(sh_out, wt_out_out)


def _densify(in_idx, in_vals, out_idx, out_vals, zeros):
    mesh = plsc.VectorSubcoreMesh(core_axis_name="c", subcore_axis_name="s",
                                  num_cores=1)
    run = functools.partial(
        pl.kernel,
        out_type=[
            jax.ShapeDtypeStruct((_WW,), jnp.float32),
            jax.ShapeDtypeStruct((_WW,), jnp.float32),
        ],
        mesh=mesh,
        scratch_types=[
            pltpu.VMEM((2 * _PER_TILE,), jnp.int32),
            pltpu.VMEM((2 * _PER_TILE,), jnp.int32),
            pltpu.VMEM((2 * _PER_TILE,), jnp.float32),
            pltpu.VMEM((2 * _NCHUNK, _CHUNK), jnp.int32),
            pltpu.VMEM_SHARED((_WW,), jnp.float32),
            pltpu.VMEM_SHARED((_WW,), jnp.float32),
            pltpu.SemaphoreType.DMA,
        ],
    )(_densify_sc)
    return run(in_idx, in_vals, out_idx, out_vals, zeros)


def _boot_body(in_idx_ref, in_vals_ref, b_in_ref, out_idx_ref, out_vals_ref,
               b_out_ref, x_ref, o_ref, wt_in_ref, wt_out_ref):
    @pl.when(pl.program_id(0) == 0)
    def _():
        rows_i = in_idx_ref[0:1, :]
        cols_i = in_idx_ref[1:2, :]
        vals_i = in_vals_ref[0:1, :]
        cmat = jnp.where(
            jax.lax.broadcasted_iota(jnp.int32, (_C, _NNZ), 0) == cols_i,
            vals_i, 0.0)
        rmat = jnp.where(
            jax.lax.broadcasted_iota(jnp.int32, (_H, _NNZ), 0) == rows_i,
            1.0, 0.0)
        wt_in_ref[...] = jax.lax.dot_general(
            cmat, rmat, (((1,), (1,)), ((), ())),
            preferred_element_type=jnp.float32)

        rows_o = out_idx_ref[0:1, :]
        cols_o = out_idx_ref[1:2, :]
        vals_o = out_vals_ref[0:1, :]
        hmat = jnp.where(
            jax.lax.broadcasted_iota(jnp.int32, (_H, _NNZ), 0) == cols_o,
            vals_o, 0.0)
        cmat2 = jnp.where(
            jax.lax.broadcasted_iota(jnp.int32, (_C, _NNZ), 0) == rows_o,
            1.0, 0.0)
        wt_out_ref[...] = jax.lax.dot_general(
            hmat, cmat2, (((1,), (1,)), ((), ())),
            preferred_element_type=jnp.float32)

    x = x_ref[...]
    h = jnp.dot(x, wt_in_ref[...], preferred_element_type=jnp.float32)
    h = jnp.maximum(h + b_in_ref[0:1, :], 0.0)
    o = jnp.dot(h, wt_out_ref[...], preferred_element_type=jnp.float32)
    o_ref[...] = o + b_out_ref[0:1, :] + x


def _main_body(prev_ref, wt_in_ref, b_in_ref, wt_out_ref, b_out_ref,
               x_ref, o_ref):
    del prev_ref
    x = x_ref[...]
    h = jnp.dot(x, wt_in_ref[...], preferred_element_type=jnp.float32)
    h = jnp.maximum(h + b_in_ref[0:1, :], 0.0)
    o = jnp.dot(h, wt_out_ref[...], preferred_element_type=jnp.float32)
    o_ref[...] = o + b_out_ref[0:1, :] + x


def kernel(x, w_in_vals, b_in, w_out_vals, b_out, in_idx, out_idx):
    zeros = jnp.zeros((_WW,), jnp.float32)
    b_in2 = b_in.reshape(1, _H)
    b_out2 = b_out.reshape(1, _C)

    wt_in_flat, wt_out_flat = _densify(
        in_idx, w_in_vals, out_idx, w_out_vals, zeros)
    wt_in = wt_in_flat.reshape(_C, _H)
    wt_out = wt_out_flat.reshape(_H, _C)

    boot = pl.pallas_call(
        _boot_body,
        grid=(_NBOOT,),
        in_specs=[
            pl.BlockSpec((2, _NNZ), lambda i: (0, 0)),
            pl.BlockSpec((1, _NNZ), lambda i: (0, 0)),
            pl.BlockSpec((1, _H), lambda i: (0, 0)),
            pl.BlockSpec((2, _NNZ), lambda i: (0, 0)),
            pl.BlockSpec((1, _NNZ), lambda i: (0, 0)),
            pl.BlockSpec((1, _C), lambda i: (0, 0)),
            pl.BlockSpec((_BLK, _C), lambda i: (i, 0)),
        ],
        out_specs=pl.BlockSpec((_BLK, _C), lambda i: (i, 0)),
        out_shape=jax.ShapeDtypeStruct((_B, _C), jnp.float32),
        scratch_shapes=[
            pltpu.VMEM((_C, _H), jnp.float32),
            pltpu.VMEM((_H, _C), jnp.float32),
        ],
    )(
        in_idx,
        w_in_vals.reshape(1, _NNZ),
        b_in2,
        out_idx,
        w_out_vals.reshape(1, _NNZ),
        b_out2,
        x,
    )

    out = pl.pallas_call(
        _main_body,
        grid=((_B - _N1) // _BLK,),
        in_specs=[
            pl.BlockSpec(memory_space=pl.ANY),
            pl.BlockSpec((_C, _H), lambda i: (0, 0)),
            pl.BlockSpec((1, _H), lambda i: (0, 0)),
            pl.BlockSpec((_H, _C), lambda i: (0, 0)),
            pl.BlockSpec((1, _C), lambda i: (0, 0)),
            pl.BlockSpec((_BLK, _C), lambda i: (i + _NBOOT, 0)),
        ],
        out_specs=pl.BlockSpec((_BLK, _C), lambda i: (i + _NBOOT, 0)),
        out_shape=jax.ShapeDtypeStruct((_B, _C), jnp.float32),
        input_output_aliases={0: 0},
    )(
        boot,
        wt_in,
        b_in2,
        wt_out,
        b_out2,
        x,
    )
    return out
